# baseline (speedup 1.0000x reference)
import functools

import jax
import jax.numpy as jnp
import numpy as np
from jax import lax
from jax.experimental import pallas as pl
from jax.experimental.pallas import tpu as pltpu
from jax.experimental.pallas import tpu_sc as plsc

_B, _T = 32, 200
_VOCAB = 100000
_D = 128
_N_SLOTS = 8192
_K = 4
_SEED = np.uint64(74565)
_GOLD = np.uint64(0x9E3779B97F4A7C15)

_TS = np.arange(0, _T - 1, 2)
_TS = _TS[_TS + 1 < _T - 1]
_P = _TS.shape[0]
_PP = 104
_PH = 56

_NC, _NS = 2, 16
_NW = _NC * _NS
_TILE_V = 33792
_NBLK = (_VOCAB + _TILE_V - 1) // _TILE_V


def _mix64(x):
    x = x ^ (x >> np.uint64(30))
    x = x * np.uint64(0xBF58476D1CE4E9B9)
    x = x ^ (x >> np.uint64(27))
    x = x * np.uint64(0x94D049BB133111EB)
    x = x ^ (x >> np.uint64(31))
    return x


def _slots_of(tok):
    h = _mix64(tok ^ _SEED)
    probe = jnp.arange(_K, dtype=jnp.uint64)
    return (_mix64(h[..., None] + probe * _GOLD) % np.uint64(_N_SLOTS)).astype(
        jnp.int32)


def _sc_gather(table, idx):
    mesh = plsc.VectorSubcoreMesh(core_axis_name="c", subcore_axis_name="s")

    @functools.partial(
        pl.kernel,
        mesh=mesh,
        out_type=jax.ShapeDtypeStruct((_B * _PP, _D), jnp.float32),
        scratch_types=[
            pltpu.VMEM((_PH,), jnp.int32),
            pltpu.VMEM((_PP - _PH,), jnp.int32),
            pltpu.VMEM((_PH, _D), jnp.float32),
            pltpu.VMEM((_PP - _PH, _D), jnp.float32),
            pltpu.SemaphoreType.DMA,
            pltpu.SemaphoreType.DMA,
        ],
    )
    def gather_kernel(table_hbm, idx_hbm, out_hbm, idx_a, idx_b, rows_a,
                      rows_b, sem_a, sem_b):
        wid = lax.axis_index("s") * _NC + lax.axis_index("c")
        base = wid * _PP
        pltpu.sync_copy(idx_hbm.at[pl.ds(base, _PH)], idx_a)
        ga = pltpu.async_copy(table_hbm.at[idx_a], rows_a, sem_a)
        pltpu.sync_copy(idx_hbm.at[pl.ds(base + _PH, _PP - _PH)], idx_b)
        gb = pltpu.async_copy(table_hbm.at[idx_b], rows_b, sem_b)
        ga.wait()
        pltpu.sync_copy(rows_a, out_hbm.at[pl.ds(base, _PH)])
        gb.wait()
        pltpu.sync_copy(rows_b, out_hbm.at[pl.ds(base + _PH, _PP - _PH)])

    return gather_kernel(table, idx)


def _tc_body(slots_ref, qslots_ref, rows_ref, w_ref, b_ref, out_ref, r_ref):
    i = pl.program_id(0)

    @pl.when(i == 0)
    def _():
        q = qslots_ref[...]
        c = jnp.zeros((_B, _PP), jnp.float32)
        for k in range(_K):
            sk = slots_ref[k]
            for kp in range(_K):
                c += (sk == q[:, kp][:, None]).astype(jnp.float32)
        rows = rows_ref[...]
        r = jnp.sum(c[:, :, None] * rows, axis=1)
        r_ref[...] = r * (1.0 / _K)

    out_ref[...] = (
        jax.lax.dot_general(
            r_ref[...], w_ref[...],
            dimension_numbers=(((1,), (1,)), ((), ())),
            preferred_element_type=jnp.float32,
        )
        + b_ref[...]
    )


def _i32(v):
    return jnp.asarray(v, dtype=jnp.int32)


def _tc_call(slots_t, qslots, rows, w, b2d):
    return pl.pallas_call(
        _tc_body,
        grid=(_NBLK,),
        in_specs=[
            pl.BlockSpec((_K, _B, _PP), lambda i: (_i32(0), _i32(0), _i32(0))),
            pl.BlockSpec((_B, _K), lambda i: (_i32(0), _i32(0))),
            pl.BlockSpec((_B, _PP, _D),
                         lambda i: (_i32(0), _i32(0), _i32(0))),
            pl.BlockSpec((_TILE_V, _D), lambda i: (i, _i32(0))),
            pl.BlockSpec((1, _TILE_V), lambda i: (_i32(0), i)),
        ],
        out_specs=pl.BlockSpec((_B, _TILE_V), lambda i: (_i32(0), i)),
        out_shape=jax.ShapeDtypeStruct((_B, _VOCAB), jnp.float32),
        scratch_shapes=[pltpu.VMEM((_B, _D), jnp.float32)],
    )(slots_t, qslots, rows, w, b2d)


def kernel(x, emb_table, W, b):
    keys = x[:, _TS].astype(jnp.uint64)
    vals = x[:, _TS + 1].astype(jnp.int32)
    slots = _slots_of(keys)
    qslots = _slots_of(x[:, -1].astype(jnp.uint64))

    slots = jnp.concatenate(
        [slots, jnp.full((_B, _PP - _P, _K), -1, jnp.int32)], axis=1)
    slots_t = slots.transpose(2, 0, 1)
    vals = jnp.concatenate(
        [vals, jnp.zeros((_B, _PP - _P), jnp.int32)], axis=1)

    rows = _sc_gather(emb_table, vals.reshape(-1))
    rows = rows.reshape(_B, _PP, _D)

    return _tc_call(slots_t, qslots, rows, W, b.reshape(1, _VOCAB))


# speedup vs baseline: 1.0114x; 1.0114x over previous
import functools

import jax
import jax.numpy as jnp
import numpy as np
from jax import lax
from jax.experimental import pallas as pl
from jax.experimental.pallas import tpu as pltpu
from jax.experimental.pallas import tpu_sc as plsc

_B, _T = 32, 200
_VOCAB = 100000
_D = 128
_N_SLOTS = 8192
_K = 4
_SEED = np.uint64(74565)
_GOLD = np.uint64(0x9E3779B97F4A7C15)

_TS = np.arange(0, _T - 1, 2)
_TS = _TS[_TS + 1 < _T - 1]
_P = _TS.shape[0]
_PP = 104
_PH = 56

_NC, _NS = 2, 16
_NW = _NC * _NS
_TILE_V = 25088
_NBLK = (_VOCAB + _TILE_V - 1) // _TILE_V


def _mix64(x):
    x = x ^ (x >> np.uint64(30))
    x = x * np.uint64(0xBF58476D1CE4E9B9)
    x = x ^ (x >> np.uint64(27))
    x = x * np.uint64(0x94D049BB133111EB)
    x = x ^ (x >> np.uint64(31))
    return x


def _slots_of(tok):
    h = _mix64(tok ^ _SEED)
    probe = jnp.arange(_K, dtype=jnp.uint64)
    return (_mix64(h[..., None] + probe * _GOLD) % np.uint64(_N_SLOTS)).astype(
        jnp.int32)


def _sc_gather(table, idx):
    mesh = plsc.VectorSubcoreMesh(core_axis_name="c", subcore_axis_name="s")

    @functools.partial(
        pl.kernel,
        mesh=mesh,
        out_type=jax.ShapeDtypeStruct((_B * _PP, _D), jnp.float32),
        scratch_types=[
            pltpu.VMEM((_PH,), jnp.int32),
            pltpu.VMEM((_PP - _PH,), jnp.int32),
            pltpu.VMEM((_PH, _D), jnp.float32),
            pltpu.VMEM((_PP - _PH, _D), jnp.float32),
            pltpu.SemaphoreType.DMA,
            pltpu.SemaphoreType.DMA,
        ],
    )
    def gather_kernel(table_hbm, idx_hbm, out_hbm, idx_a, idx_b, rows_a,
                      rows_b, sem_a, sem_b):
        wid = lax.axis_index("s") * _NC + lax.axis_index("c")
        base = wid * _PP
        pltpu.sync_copy(idx_hbm.at[pl.ds(base, _PH)], idx_a)
        ga = pltpu.async_copy(table_hbm.at[idx_a], rows_a, sem_a)
        pltpu.sync_copy(idx_hbm.at[pl.ds(base + _PH, _PP - _PH)], idx_b)
        gb = pltpu.async_copy(table_hbm.at[idx_b], rows_b, sem_b)
        ga.wait()
        pltpu.sync_copy(rows_a, out_hbm.at[pl.ds(base, _PH)])
        gb.wait()
        pltpu.sync_copy(rows_b, out_hbm.at[pl.ds(base + _PH, _PP - _PH)])

    return gather_kernel(table, idx)


def _tc_body(slots_ref, qslots_ref, rows_ref, w_ref, b_ref, out_ref, r_ref):
    i = pl.program_id(0)

    @pl.when(i == 0)
    def _():
        q = qslots_ref[...]
        c = jnp.zeros((_B, _PP), jnp.float32)
        for k in range(_K):
            sk = slots_ref[k]
            for kp in range(_K):
                c += (sk == q[:, kp][:, None]).astype(jnp.float32)
        rows = rows_ref[...]
        r = jnp.sum(c[:, :, None] * rows, axis=1)
        r_ref[...] = r * (1.0 / _K)

    out_ref[...] = (
        jax.lax.dot_general(
            r_ref[...], w_ref[...],
            dimension_numbers=(((1,), (1,)), ((), ())),
            preferred_element_type=jnp.float32,
        )
        + b_ref[...]
    )


def _i32(v):
    return jnp.asarray(v, dtype=jnp.int32)


def _tc_call(slots_t, qslots, rows, w, b2d):
    return pl.pallas_call(
        _tc_body,
        grid=(_NBLK,),
        in_specs=[
            pl.BlockSpec((_K, _B, _PP), lambda i: (_i32(0), _i32(0), _i32(0))),
            pl.BlockSpec((_B, _K), lambda i: (_i32(0), _i32(0))),
            pl.BlockSpec((_B, _PP, _D),
                         lambda i: (_i32(0), _i32(0), _i32(0))),
            pl.BlockSpec((_TILE_V, _D), lambda i: (i, _i32(0))),
            pl.BlockSpec((1, _TILE_V), lambda i: (_i32(0), i)),
        ],
        out_specs=pl.BlockSpec((_B, _TILE_V), lambda i: (_i32(0), i)),
        out_shape=jax.ShapeDtypeStruct((_B, _VOCAB), jnp.float32),
        scratch_shapes=[pltpu.VMEM((_B, _D), jnp.float32)],
    )(slots_t, qslots, rows, w, b2d)


def kernel(x, emb_table, W, b):
    keys = x[:, _TS].astype(jnp.uint64)
    vals = x[:, _TS + 1].astype(jnp.int32)
    slots = _slots_of(keys)
    qslots = _slots_of(x[:, -1].astype(jnp.uint64))

    slots = jnp.concatenate(
        [slots, jnp.full((_B, _PP - _P, _K), -1, jnp.int32)], axis=1)
    slots_t = slots.transpose(2, 0, 1)
    vals = jnp.concatenate(
        [vals, jnp.zeros((_B, _PP - _P), jnp.int32)], axis=1)

    rows = _sc_gather(emb_table, vals.reshape(-1))
    rows = rows.reshape(_B, _PP, _D)

    return _tc_call(slots_t, qslots, rows, W, b.reshape(1, _VOCAB))
